# baseline (device time: 39401 ns/iter reference)
import jax
import jax.numpy as jnp
from jax import lax
from jax.experimental import pallas as pl
from jax.experimental.pallas import tpu as pltpu

B, S, H, Dh, Dr = 2, 256, 16, 64, 32
D = 1024
DC_SH = 64
BS = B * S
SCALE = (Dh + Dr) ** -0.5


def kernel(x, Wdkv, Wuk, Wuv, Wq, Wqr, Wkr, Wo):
    def body(x_ref, wdkv_ref, wuk_ref, wuv_ref, wq_ref, wqr_ref, wkr_ref,
             wo_ref, out_ref,
             c_send, c_recv, wuk_send, wuk_recv, wuv_send, wuv_recv,
             obuf, send_sems, recv_sems):
        my_x = lax.axis_index("x")
        my_y = lax.axis_index("y")
        nbr = (my_x, 1 - my_y)

        barrier = pltpu.get_barrier_semaphore()
        pl.semaphore_signal(barrier, inc=1, device_id=nbr,
                            device_id_type=pl.DeviceIdType.MESH)
        pl.semaphore_wait(barrier, 1)

        bf16 = jnp.bfloat16
        f32 = jnp.float32

        wuk_send[...] = wuk_ref[...].astype(bf16)
        wuv_send[...] = wuv_ref[...].astype(bf16)
        rdma_wuk = pltpu.make_async_remote_copy(
            src_ref=wuk_send, dst_ref=wuk_recv,
            send_sem=send_sems.at[0], recv_sem=recv_sems.at[0],
            device_id=nbr, device_id_type=pl.DeviceIdType.MESH)
        rdma_wuk.start()
        rdma_wuv = pltpu.make_async_remote_copy(
            src_ref=wuv_send, dst_ref=wuv_recv,
            send_sem=send_sems.at[1], recv_sem=recv_sems.at[1],
            device_id=nbr, device_id_type=pl.DeviceIdType.MESH)
        rdma_wuv.start()

        xb = x_ref[...].reshape(BS, D).astype(bf16)
        c_send[...] = jnp.dot(xb, wdkv_ref[...].astype(bf16),
                              preferred_element_type=f32).astype(bf16)
        rdma_c = pltpu.make_async_remote_copy(
            src_ref=c_send, dst_ref=c_recv,
            send_sem=send_sems.at[2], recv_sem=recv_sems.at[2],
            device_id=nbr, device_id_type=pl.DeviceIdType.MESH)
        rdma_c.start()

        Q = jnp.dot(xb, wq_ref[...].astype(bf16),
                    preferred_element_type=f32).astype(bf16)
        Qr = jnp.dot(xb, wqr_ref[...].astype(bf16),
                     preferred_element_type=f32).astype(bf16)
        Kr = jnp.dot(xb, wkr_ref[...].astype(bf16),
                     preferred_element_type=f32).astype(bf16)

        rdma_wuk.wait_recv()
        rdma_wuv.wait_recv()
        rdma_c.wait_recv()

        c_loc = c_send[...]
        c_rem = c_recv[...]
        K = (jnp.dot(c_loc, wuk_send[...], preferred_element_type=f32)
             + jnp.dot(c_rem, wuk_recv[...], preferred_element_type=f32)
             ).astype(bf16)
        V = (jnp.dot(c_loc, wuv_send[...], preferred_element_type=f32)
             + jnp.dot(c_rem, wuv_recv[...], preferred_element_type=f32)
             ).astype(bf16)

        for b in range(B):
            r0 = b * S
            Krb = Kr[r0:r0 + S, :]
            for h in range(H):
                Qh = Q[r0:r0 + S, h * Dh:(h + 1) * Dh]
                Kh = K[r0:r0 + S, h * Dh:(h + 1) * Dh]
                Vh = V[r0:r0 + S, h * Dh:(h + 1) * Dh]
                Qrh = Qr[r0:r0 + S, h * Dr:(h + 1) * Dr]
                s = lax.dot_general(Qh, Kh, (((1,), (1,)), ((), ())),
                                    preferred_element_type=f32)
                s = s + lax.dot_general(Qrh, Krb, (((1,), (1,)), ((), ())),
                                        preferred_element_type=f32)
                s = s * SCALE
                m = jnp.max(s, axis=1, keepdims=True)
                p = jnp.exp(s - m)
                p = p / jnp.sum(p, axis=1, keepdims=True)
                oh = jnp.dot(p.astype(bf16), Vh, preferred_element_type=f32)
                obuf[r0:r0 + S, h * Dh:(h + 1) * Dh] = oh.astype(bf16)

        out = jnp.dot(obuf[...], wo_ref[...].astype(bf16),
                      preferred_element_type=f32)
        out_ref[...] = out.reshape(B, S, D)

        rdma_wuk.wait_send()
        rdma_wuv.wait_send()
        rdma_c.wait_send()

    return pl.pallas_call(
        body,
        out_shape=jax.ShapeDtypeStruct((B, S, D), jnp.float32),
        in_specs=[pl.BlockSpec(memory_space=pltpu.VMEM)] * 8,
        out_specs=pl.BlockSpec(memory_space=pltpu.VMEM),
        scratch_shapes=[
            pltpu.VMEM((BS, DC_SH), jnp.bfloat16),
            pltpu.VMEM((BS, DC_SH), jnp.bfloat16),
            pltpu.VMEM((DC_SH, D), jnp.bfloat16),
            pltpu.VMEM((DC_SH, D), jnp.bfloat16),
            pltpu.VMEM((DC_SH, D), jnp.bfloat16),
            pltpu.VMEM((DC_SH, D), jnp.bfloat16),
            pltpu.VMEM((BS, H * Dh), jnp.bfloat16),
            pltpu.SemaphoreType.DMA((3,)),
            pltpu.SemaphoreType.DMA((3,)),
        ],
        compiler_params=pltpu.CompilerParams(collective_id=0),
    )(x, Wdkv, Wuk, Wuv, Wq, Wqr, Wkr, Wo)
